# Initial kernel scaffold; baseline (speedup 1.0000x reference)
import jax
import jax.numpy as jnp
from jax.experimental import pallas as pl


def kernel(feat_user, feat_item, eidx_rates, eidx_rated_by, eidx_follows, eidx_similar, W_micro_user, W_micro_item, A_micro_user, A_micro_item, W_node_user, W_node_item, W_rel_rates, W_rel_rated_by, W_rel_follows, W_rel_similar, macro_attn, W_res_user, b_res_user, W_res_item, b_res_item, rw_user, rw_item):
    raise NotImplementedError("write your pallas kernel here")


# speedup vs baseline: 6.8604x; 6.8604x over previous
import dataclasses
import functools

import jax
import jax.numpy as jnp
from jax import lax
from jax.experimental import pallas as pl
from jax.experimental.pallas import tpu as pltpu
from jax.experimental.pallas import tpu_sc as plsc

N = 50000
E = 150000
D_IN = 128
K = 4
D_OUT = 32
HID = K * D_OUT

NSC = 16
NCORE = 2
CHUNK = 32

NP = 50176
ROWS_PER_SUB = NP // NSC
ZROWS = 64

EP = 151552
EW_A = EP // (NCORE * NSC)
NCH_A = EW_A // CHUNK
EW_C = EP // NSC
NCH_C = EW_C // CHUNK

BN = 512
NBLK = NP // BN

@functools.lru_cache(maxsize=None)
def _sc_params():
    cp = pltpu.CompilerParams()
    if "needs_layout_passes" in pltpu.CompilerParams.__dataclass_fields__:
        cp = dataclasses.replace(cp, needs_layout_passes=False)
    return cp


@functools.lru_cache(maxsize=None)
def _vector_mesh():
    return plsc.VectorSubcoreMesh(core_axis_name="c", subcore_axis_name="s",
                                  num_cores=NCORE, num_subcores=NSC)


def _pre_body(fu_ref, fi_ref, wmu_ref, wmi_ref, au_ref, ai_ref,
              hu_ref, hi_ref, elr_ref):
    fu = fu_ref[...]
    fi = fi_ref[...]
    hu = jnp.dot(fu, wmu_ref[...], preferred_element_type=jnp.float32)
    hi = jnp.dot(fi, wmi_ref[...], preferred_element_type=jnp.float32)
    hu_ref[...] = hu
    hi_ref[...] = hi
    au = au_ref[...]
    ai = ai_ref[...]

    def tab(h, a, lo):
        cols = []
        for k in range(K):
            hk = h[:, k * D_OUT:(k + 1) * D_OUT]
            ak = a[k, lo:lo + D_OUT][None, :]
            cols.append((hk * ak).sum(axis=1, keepdims=True))
        return jnp.concatenate(cols, axis=1)

    z8 = jnp.zeros((fu.shape[0], 8), jnp.float32)
    z96 = jnp.zeros((fu.shape[0], 96), jnp.float32)
    elr_ref[...] = jnp.concatenate(
        [tab(hu, au, 0), tab(hi, ai, 0), z8,
         tab(hu, au, D_OUT), tab(hi, au, D_OUT),
         tab(hu, ai, D_OUT), tab(hi, ai, D_OUT), z96], axis=1)


def _pre_call(fu, fi, wmu, wmi, au, ai):
    row_spec = pl.BlockSpec((BN, D_IN), lambda i: (i, 0))
    full_spec = pl.BlockSpec((D_IN, HID), lambda i: (0, 0))
    a_spec = pl.BlockSpec((K, 2 * D_OUT), lambda i: (0, 0))
    h_out = jax.ShapeDtypeStruct((NP, HID), jnp.float32)
    return pl.pallas_call(
        _pre_body,
        grid=(NBLK,),
        in_specs=[row_spec, row_spec, full_spec, full_spec, a_spec, a_spec],
        out_specs=[row_spec, row_spec, row_spec],
        out_shape=[h_out, h_out, h_out],
    )(fu, fi, wmu, wmi, au, ai)


NP8 = NP // 8
NP4 = NP // 4


def _passA_body(el_off, er_off, src_hbm, dst_hbm, elr_hbm, z_hbm, ee_hbm,
                esum_hbm, sbuf, dbuf, dbuf2, srcrow, dstrow, eebuf, obuf,
                esum_sh):
    c = lax.axis_index("c")
    s = lax.axis_index("s")
    w = s * NCORE + c

    @pl.when(s == 0)
    def _zero():
        pltpu.sync_copy(z_hbm, esum_sh)

    plsc.subcore_barrier()

    base = w * EW_A

    @pl.loop(0, NCH_A)
    def _edges(i):
        off = base + i * CHUNK
        pltpu.sync_copy(src_hbm.at[pl.ds(off, CHUNK)], sbuf)
        pltpu.sync_copy(dst_hbm.at[pl.ds(off, CHUNK)], dbuf)
        pltpu.sync_copy(elr_hbm.at[sbuf], srcrow)
        pltpu.sync_copy(elr_hbm.at[dbuf], dstrow)

        @pl.loop(0, CHUNK // 16)
        def _d2(g):
            dv = dbuf.at[pl.ds(g * 16, 16)][...]
            dbuf2.at[pl.ds(g * 16, 16)][...] = lax.shift_right_logical(dv, 3)

        @pl.loop(0, CHUNK)
        def _ee(j):
            lane = lax.iota(jnp.int32, 16) & 3
            jv = jnp.full((16,), j, jnp.int32)
            el = plsc.load_gather(srcrow, [jv, lane + el_off])
            er = plsc.load_gather(dstrow, [jv, lane + (16 + er_off)])
            x = el + er
            e = jnp.maximum(x, 0.2 * x)
            ee = jnp.exp(e)
            eebuf.at[j][...] = ee
            dv = plsc.load_gather(dbuf, [jv])
            grp = dv & 7
            for p in range(8):
                obuf.at[j, pl.ds(16 * p, 16)][...] = jnp.where(
                    grp == p, ee, 0.0)

        pltpu.sync_copy(eebuf, ee_hbm.at[pl.ds(off, CHUNK)])
        pltpu.sync_copy(obuf, esum_sh.at[dbuf2], add=True)

    plsc.subcore_barrier()

    @pl.when(s == 0)
    def _flush():
        pltpu.sync_copy(esum_sh, esum_hbm.at[pl.ds(c * NP8, NP8)])


@functools.lru_cache(maxsize=None)
def _passA(el_off, er_off):
    return pl.kernel(
        functools.partial(_passA_body, el_off, er_off),
        out_type=(jax.ShapeDtypeStruct((EP, 16), jnp.float32),
                  jax.ShapeDtypeStruct((NCORE * NP8, HID), jnp.float32)),
        mesh=_vector_mesh(),
        scratch_types=[
            pltpu.VMEM((CHUNK,), jnp.int32),
            pltpu.VMEM((CHUNK,), jnp.int32),
            pltpu.VMEM((CHUNK,), jnp.int32),
            pltpu.VMEM((CHUNK, HID), jnp.float32),
            pltpu.VMEM((CHUNK, HID), jnp.float32),
            pltpu.VMEM((CHUNK, 16), jnp.float32),
            pltpu.VMEM((CHUNK, HID), jnp.float32),
            pltpu.VMEM_SHARED((NP8, HID), jnp.float32),
        ],
        compiler_params=_sc_params(),
    )


def _passC_body(src_hbm, dst_hbm, h_hbm, ee_hbm, z_hbm, ft_hbm,
                sbuf, dbuf, dbuf2, hbuf, eebuf, obuf, acc_sh):
    c = lax.axis_index("c")
    s = lax.axis_index("s")

    for h in range(K):
        @pl.when(c == h // 2)
        def _head():
            @pl.when(s == 0)
            def _zero():
                pltpu.sync_copy(z_hbm, acc_sh)

            plsc.subcore_barrier()

            @pl.loop(0, NCH_C)
            def _edges(i):
                off = s * EW_C + i * CHUNK
                pltpu.sync_copy(src_hbm.at[pl.ds(off, CHUNK)], sbuf)
                pltpu.sync_copy(dst_hbm.at[pl.ds(off, CHUNK)], dbuf)
                pltpu.sync_copy(h_hbm.at[sbuf], hbuf)
                pltpu.sync_copy(ee_hbm.at[pl.ds(off, CHUNK)], eebuf)

                @pl.loop(0, CHUNK // 16)
                def _d2(g):
                    dv = dbuf.at[pl.ds(g * 16, 16)][...]
                    dbuf2.at[pl.ds(g * 16, 16)][...] = (
                        lax.shift_right_logical(dv, 2))

                @pl.loop(0, CHUNK)
                def _scale(j):
                    jv = jnp.full((16,), j, jnp.int32)
                    hv = jnp.full((16,), h, jnp.int32)
                    a = plsc.load_gather(eebuf, [jv, hv])
                    v0 = hbuf.at[j, pl.ds(32 * h, 16)][...] * a
                    v1 = hbuf.at[j, pl.ds(32 * h + 16, 16)][...] * a
                    dv = plsc.load_gather(dbuf, [jv])
                    grp = dv & 3
                    for p in range(4):
                        obuf.at[j, pl.ds(32 * p, 16)][...] = jnp.where(
                            grp == p, v0, 0.0)
                        obuf.at[j, pl.ds(32 * p + 16, 16)][...] = jnp.where(
                            grp == p, v1, 0.0)

                pltpu.sync_copy(obuf, acc_sh.at[dbuf2], add=True)

            plsc.subcore_barrier()

            @pl.when(s == 0)
            def _flush():
                pltpu.sync_copy(acc_sh, ft_hbm.at[pl.ds(h * NP4, NP4)])

            plsc.subcore_barrier()


@functools.lru_cache(maxsize=None)
def _passC():
    return pl.kernel(
        _passC_body,
        out_type=jax.ShapeDtypeStruct((K * NP4, HID), jnp.float32),
        mesh=_vector_mesh(),
        scratch_types=[
            pltpu.VMEM((CHUNK,), jnp.int32),
            pltpu.VMEM((CHUNK,), jnp.int32),
            pltpu.VMEM((CHUNK,), jnp.int32),
            pltpu.VMEM((CHUNK, HID), jnp.float32),
            pltpu.VMEM((CHUNK, 16), jnp.float32),
            pltpu.VMEM((CHUNK, HID), jnp.float32),
            pltpu.VMEM_SHARED((NP4, HID), jnp.float32),
        ],
        compiler_params=_sc_params(),
    )


def _post_body(f_ref, fta0, fta1, fta2, fta3, ftb0, ftb1, ftb2, ftb3,
               esa0, esa1, esb0, esb1, wrela_ref, wrelb_ref, wn_ref,
               wres_ref, bres_ref, ma_ref, alpha_ref, out_ref):
    f = f_ref[...]

    def zrel(ft_refs, es0, es1, wrel_ref):
        esum = es0[...] + es1[...]
        cols = []
        for k in range(K):
            d = esum[:, k:k + 1]
            d = jnp.where(d > 0.0, d, 1.0)
            cols.append(jnp.maximum(ft_refs[k][...] / d, 0.0))
        r = jnp.concatenate(cols, axis=1)
        return jnp.dot(r, wrel_ref[...], preferred_element_type=jnp.float32)

    za = zrel((fta0, fta1, fta2, fta3), esa0, esa1, wrela_ref)
    zb = zrel((ftb0, ftb1, ftb2, ftb3), esb0, esb1, wrelb_ref)
    node = jnp.dot(f, wn_ref[...], preferred_element_type=jnp.float32)
    ma = ma_ref[...]

    outs = []
    for k in range(K):
        sl = slice(k * D_OUT, (k + 1) * D_OUT)
        nk, zak, zbk = node[:, sl], za[:, sl], zb[:, sl]
        nbase = (nk * ma[k, :D_OUT][None, :]).sum(axis=1, keepdims=True)
        sa = nbase + (zak * ma[k, D_OUT:][None, :]).sum(axis=1, keepdims=True)
        sb = nbase + (zbk * ma[k, D_OUT:][None, :]).sum(axis=1, keepdims=True)
        sa = jnp.maximum(sa, 0.2 * sa)
        sb = jnp.maximum(sb, 0.2 * sb)
        m = jnp.maximum(sa, sb)
        pa = jnp.exp(sa - m)
        pb = jnp.exp(sb - m)
        outs.append((pa * zak + pb * zbk) / (pa + pb))
    mac = jnp.concatenate(outs, axis=1)

    res = jnp.dot(f, wres_ref[...], preferred_element_type=jnp.float32) \
        + bres_ref[...]
    al = alpha_ref[0, 0]
    out_ref[...] = al * mac + (1.0 - al) * res


def _post_call(f, fta, ftb, esa, esb, wrela, wrelb, wn, wres, bres, ma, alpha):
    row_spec = pl.BlockSpec((BN, D_IN), lambda i: (i, 0))
    ft_specs_a = [pl.BlockSpec((BN, D_OUT), lambda i, k=k: (k * NBLK + i, 0))
                  for k in range(K)]
    ft_specs_b = [pl.BlockSpec((BN, D_OUT), lambda i, k=k: (k * NBLK + i, 0))
                  for k in range(K)]
    es_specs_a = [pl.BlockSpec((BN, 16), lambda i, c=c: (c * NBLK + i, 0))
                  for c in range(NCORE)]
    es_specs_b = [pl.BlockSpec((BN, 16), lambda i, c=c: (c * NBLK + i, 0))
                  for c in range(NCORE)]
    w_spec = pl.BlockSpec((HID, HID), lambda i: (0, 0))
    b_spec = pl.BlockSpec((1, HID), lambda i: (0, 0))
    ma_spec = pl.BlockSpec((K, 2 * D_OUT), lambda i: (0, 0))
    al_spec = pl.BlockSpec((1, 1), lambda i: (0, 0))
    return pl.pallas_call(
        _post_body,
        grid=(NBLK,),
        in_specs=([row_spec] + [ft_specs_a[k] for k in range(K)]
                  + [ft_specs_b[k] for k in range(K)]
                  + es_specs_a + es_specs_b
                  + [w_spec, w_spec, w_spec, w_spec, b_spec, ma_spec, al_spec]),
        out_specs=row_spec,
        out_shape=jax.ShapeDtypeStruct((NP, D_IN), jnp.float32),
    )(f, fta, fta, fta, fta, ftb, ftb, ftb, ftb, esa, esa, esb, esb,
      wrela, wrelb, wn, wres, bres, ma, alpha)


def kernel(feat_user, feat_item, eidx_rates, eidx_rated_by, eidx_follows,
           eidx_similar, W_micro_user, W_micro_item, A_micro_user,
           A_micro_item, W_node_user, W_node_item, W_rel_rates,
           W_rel_rated_by, W_rel_follows, W_rel_similar, macro_attn,
           W_res_user, b_res_user, W_res_item, b_res_item, rw_user, rw_item):
    fu = jnp.pad(feat_user, ((0, NP - N), (0, 0)))
    fi = jnp.pad(feat_item, ((0, NP - N), (0, 0)))

    def pad_edges(eidx):
        src = jnp.pad(eidx[0].astype(jnp.int32), (0, EP - E),
                      constant_values=N)
        dst = jnp.pad(eidx[1].astype(jnp.int32), (0, EP - E),
                      constant_values=N)
        return src, dst

    src_rt, dst_rt = pad_edges(eidx_rates)
    src_rb, dst_rb = pad_edges(eidx_rated_by)
    src_fo, dst_fo = pad_edges(eidx_follows)
    src_si, dst_si = pad_edges(eidx_similar)

    hu, hi, elr = _pre_call(
        fu, fi, W_micro_user, W_micro_item, A_micro_user, A_micro_item)

    z8p = jnp.zeros((NP8, HID), jnp.float32)
    z4p = jnp.zeros((NP4, HID), jnp.float32)

    ee_rt, es_rt = _passA(0, 4)(src_rt, dst_rt, elr, z8p)
    ee_rb, es_rb = _passA(4, 8)(src_rb, dst_rb, elr, z8p)
    ee_fo, es_fo = _passA(0, 0)(src_fo, dst_fo, elr, z8p)
    ee_si, es_si = _passA(4, 12)(src_si, dst_si, elr, z8p)

    ft_rt = _passC()(src_rt, dst_rt, hu, ee_rt, z4p)
    ft_rb = _passC()(src_rb, dst_rb, hi, ee_rb, z4p)
    ft_fo = _passC()(src_fo, dst_fo, hu, ee_fo, z4p)
    ft_si = _passC()(src_si, dst_si, hi, ee_si, z4p)

    def unpack_es(es):
        return es.reshape(NCORE * NP, 16)

    def unpack_ft(ft):
        return ft.reshape(K * NP, D_OUT)

    es_rt, es_rb, es_fo, es_si = map(unpack_es, (es_rt, es_rb, es_fo, es_si))
    ft_rt, ft_rb, ft_fo, ft_si = map(unpack_ft, (ft_rt, ft_rb, ft_fo, ft_si))

    au = jax.nn.sigmoid(rw_user).reshape(1, 1)
    ai = jax.nn.sigmoid(rw_item).reshape(1, 1)

    out_user = _post_call(fu, ft_rb, ft_fo, es_rb, es_fo, W_rel_rated_by,
                          W_rel_follows, W_node_user, W_res_user,
                          b_res_user.reshape(1, HID), macro_attn, au)
    out_item = _post_call(fi, ft_rt, ft_si, es_rt, es_si, W_rel_rates,
                          W_rel_similar, W_node_item, W_res_item,
                          b_res_item.reshape(1, HID), macro_attn, ai)
    return out_user[:N], out_item[:N]


# speedup vs baseline: 9.0717x; 1.3223x over previous
import dataclasses
import functools

import jax
import jax.numpy as jnp
from jax import lax
from jax.experimental import pallas as pl
from jax.experimental.pallas import tpu as pltpu
from jax.experimental.pallas import tpu_sc as plsc

N = 50000
E = 150000
D_IN = 128
K = 4
D_OUT = 32
HID = K * D_OUT

NSC = 16
NCORE = 2
CHUNK = 64

NP = 50176
ROWS_PER_SUB = NP // NSC
ZROWS = 64

EP = 151552
EW_A = EP // (NCORE * NSC)
NCH_A = EW_A // CHUNK
EW_C = EP // NSC
NCH_C = EW_C // CHUNK

BN = 512
NBLK = NP // BN

@functools.lru_cache(maxsize=None)
def _sc_params():
    cp = pltpu.CompilerParams()
    if "needs_layout_passes" in pltpu.CompilerParams.__dataclass_fields__:
        cp = dataclasses.replace(cp, needs_layout_passes=False)
    return cp


@functools.lru_cache(maxsize=None)
def _vector_mesh():
    return plsc.VectorSubcoreMesh(core_axis_name="c", subcore_axis_name="s",
                                  num_cores=NCORE, num_subcores=NSC)


def _pre_body(fu_ref, fi_ref, wmu_ref, wmi_ref, au_ref, ai_ref,
              hu_ref, hi_ref, elr_ref):
    fu = fu_ref[...]
    fi = fi_ref[...]
    hu = jnp.dot(fu, wmu_ref[...], preferred_element_type=jnp.float32)
    hi = jnp.dot(fi, wmi_ref[...], preferred_element_type=jnp.float32)
    hu_ref[...] = hu
    hi_ref[...] = hi
    au = au_ref[...]
    ai = ai_ref[...]

    def tab(h, a, lo):
        cols = []
        for k in range(K):
            hk = h[:, k * D_OUT:(k + 1) * D_OUT]
            ak = a[k, lo:lo + D_OUT][None, :]
            cols.append((hk * ak).sum(axis=1, keepdims=True))
        return jnp.concatenate(cols, axis=1)

    z8 = jnp.zeros((fu.shape[0], 8), jnp.float32)
    z96 = jnp.zeros((fu.shape[0], 96), jnp.float32)
    elr_ref[...] = jnp.concatenate(
        [tab(hu, au, 0), tab(hi, ai, 0), z8,
         tab(hu, au, D_OUT), tab(hi, au, D_OUT),
         tab(hu, ai, D_OUT), tab(hi, ai, D_OUT), z96], axis=1)


def _pre_call(fu, fi, wmu, wmi, au, ai):
    row_spec = pl.BlockSpec((BN, D_IN), lambda i: (i, 0))
    full_spec = pl.BlockSpec((D_IN, HID), lambda i: (0, 0))
    a_spec = pl.BlockSpec((K, 2 * D_OUT), lambda i: (0, 0))
    h_out = jax.ShapeDtypeStruct((NP, HID), jnp.float32)
    return pl.pallas_call(
        _pre_body,
        grid=(NBLK,),
        in_specs=[row_spec, row_spec, full_spec, full_spec, a_spec, a_spec],
        out_specs=[row_spec, row_spec, row_spec],
        out_shape=[h_out, h_out, h_out],
    )(fu, fi, wmu, wmi, au, ai)


NP8 = NP // 8
NP4 = NP // 4


def _passA_body(el_off, er_off, src_hbm, dst_hbm, elr_hbm, z_hbm, ee_hbm,
                esum_hbm, sbuf, dbuf, dbuf2, srcrow, dstrow, eebuf, obuf,
                esum_sh):
    c = lax.axis_index("c")
    s = lax.axis_index("s")
    w = s * NCORE + c

    @pl.when(s == 0)
    def _zero():
        pltpu.sync_copy(z_hbm, esum_sh)

    plsc.subcore_barrier()

    base = w * EW_A

    @pl.loop(0, NCH_A)
    def _edges(i):
        off = base + i * CHUNK
        pltpu.sync_copy(src_hbm.at[pl.ds(off, CHUNK)], sbuf)
        pltpu.sync_copy(dst_hbm.at[pl.ds(off, CHUNK)], dbuf)
        pltpu.sync_copy(elr_hbm.at[sbuf], srcrow)
        pltpu.sync_copy(elr_hbm.at[dbuf], dstrow)

        @pl.loop(0, CHUNK // 16)
        def _d2(g):
            dv = dbuf.at[pl.ds(g * 16, 16)][...]
            dbuf2.at[pl.ds(g * 16, 16)][...] = lax.shift_right_logical(dv, 3)

        @pl.loop(0, CHUNK)
        def _ee(j):
            lane = lax.iota(jnp.int32, 16) & 3
            jv = jnp.full((16,), j, jnp.int32)
            el = plsc.load_gather(srcrow, [jv, lane + el_off])
            er = plsc.load_gather(dstrow, [jv, lane + (16 + er_off)])
            x = el + er
            e = jnp.maximum(x, 0.2 * x)
            ee = jnp.exp(e)
            eebuf.at[j][...] = ee
            dv = plsc.load_gather(dbuf, [jv])
            grp = dv & 7
            for p in range(8):
                obuf.at[j, pl.ds(16 * p, 16)][...] = jnp.where(
                    grp == p, ee, 0.0)

        pltpu.sync_copy(eebuf, ee_hbm.at[pl.ds(off, CHUNK)])
        pltpu.sync_copy(obuf, esum_sh.at[dbuf2], add=True)

    plsc.subcore_barrier()

    @pl.when(s == 0)
    def _flush():
        pltpu.sync_copy(esum_sh, esum_hbm.at[pl.ds(c * NP8, NP8)])


@functools.lru_cache(maxsize=None)
def _passA(el_off, er_off):
    return pl.kernel(
        functools.partial(_passA_body, el_off, er_off),
        out_type=(jax.ShapeDtypeStruct((EP, 16), jnp.float32),
                  jax.ShapeDtypeStruct((NCORE * NP8, HID), jnp.float32)),
        mesh=_vector_mesh(),
        scratch_types=[
            pltpu.VMEM((CHUNK,), jnp.int32),
            pltpu.VMEM((CHUNK,), jnp.int32),
            pltpu.VMEM((CHUNK,), jnp.int32),
            pltpu.VMEM((CHUNK, HID), jnp.float32),
            pltpu.VMEM((CHUNK, HID), jnp.float32),
            pltpu.VMEM((CHUNK, 16), jnp.float32),
            pltpu.VMEM((CHUNK, HID), jnp.float32),
            pltpu.VMEM_SHARED((NP8, HID), jnp.float32),
        ],
        compiler_params=_sc_params(),
    )


def _passC_body(src_hbm, dst_hbm, h_hbm, ee_hbm, z_hbm, ft_hbm,
                sbuf, dbuf, dbuf2, hbuf, eebuf, obuf, acc_sh):
    c = lax.axis_index("c")
    s = lax.axis_index("s")

    for h in range(K):
        @pl.when(c == h // 2)
        def _head():
            @pl.when(s == 0)
            def _zero():
                pltpu.sync_copy(z_hbm, acc_sh)

            plsc.subcore_barrier()

            @pl.loop(0, NCH_C)
            def _edges(i):
                off = s * EW_C + i * CHUNK
                pltpu.sync_copy(src_hbm.at[pl.ds(off, CHUNK)], sbuf)
                pltpu.sync_copy(dst_hbm.at[pl.ds(off, CHUNK)], dbuf)
                pltpu.sync_copy(h_hbm.at[sbuf], hbuf)
                pltpu.sync_copy(ee_hbm.at[pl.ds(off, CHUNK)], eebuf)

                @pl.loop(0, CHUNK // 16)
                def _d2(g):
                    dv = dbuf.at[pl.ds(g * 16, 16)][...]
                    dbuf2.at[pl.ds(g * 16, 16)][...] = (
                        lax.shift_right_logical(dv, 2))

                @pl.loop(0, CHUNK)
                def _scale(j):
                    jv = jnp.full((16,), j, jnp.int32)
                    hv = jnp.full((16,), h, jnp.int32)
                    a = plsc.load_gather(eebuf, [jv, hv])
                    v0 = hbuf.at[j, pl.ds(32 * h, 16)][...] * a
                    v1 = hbuf.at[j, pl.ds(32 * h + 16, 16)][...] * a
                    dv = plsc.load_gather(dbuf, [jv])
                    grp = dv & 3
                    for p in range(4):
                        obuf.at[j, pl.ds(32 * p, 16)][...] = jnp.where(
                            grp == p, v0, 0.0)
                        obuf.at[j, pl.ds(32 * p + 16, 16)][...] = jnp.where(
                            grp == p, v1, 0.0)

                pltpu.sync_copy(obuf, acc_sh.at[dbuf2], add=True)

            plsc.subcore_barrier()

            @pl.when(s == 0)
            def _flush():
                pltpu.sync_copy(acc_sh, ft_hbm.at[pl.ds(h * NP4, NP4)])

            plsc.subcore_barrier()


@functools.lru_cache(maxsize=None)
def _passC():
    return pl.kernel(
        _passC_body,
        out_type=jax.ShapeDtypeStruct((K * NP4, HID), jnp.float32),
        mesh=_vector_mesh(),
        scratch_types=[
            pltpu.VMEM((CHUNK,), jnp.int32),
            pltpu.VMEM((CHUNK,), jnp.int32),
            pltpu.VMEM((CHUNK,), jnp.int32),
            pltpu.VMEM((CHUNK, HID), jnp.float32),
            pltpu.VMEM((CHUNK, 16), jnp.float32),
            pltpu.VMEM((CHUNK, HID), jnp.float32),
            pltpu.VMEM_SHARED((NP4, HID), jnp.float32),
        ],
        compiler_params=_sc_params(),
    )


def _post_body(f_ref, fta0, fta1, fta2, fta3, ftb0, ftb1, ftb2, ftb3,
               esa0, esa1, esb0, esb1, wrela_ref, wrelb_ref, wn_ref,
               wres_ref, bres_ref, ma_ref, alpha_ref, out_ref):
    f = f_ref[...]

    def zrel(ft_refs, es0, es1, wrel_ref):
        esum = es0[...] + es1[...]
        cols = []
        for k in range(K):
            d = esum[:, k:k + 1]
            d = jnp.where(d > 0.0, d, 1.0)
            cols.append(jnp.maximum(ft_refs[k][...] / d, 0.0))
        r = jnp.concatenate(cols, axis=1)
        return jnp.dot(r, wrel_ref[...], preferred_element_type=jnp.float32)

    za = zrel((fta0, fta1, fta2, fta3), esa0, esa1, wrela_ref)
    zb = zrel((ftb0, ftb1, ftb2, ftb3), esb0, esb1, wrelb_ref)
    node = jnp.dot(f, wn_ref[...], preferred_element_type=jnp.float32)
    ma = ma_ref[...]

    outs = []
    for k in range(K):
        sl = slice(k * D_OUT, (k + 1) * D_OUT)
        nk, zak, zbk = node[:, sl], za[:, sl], zb[:, sl]
        nbase = (nk * ma[k, :D_OUT][None, :]).sum(axis=1, keepdims=True)
        sa = nbase + (zak * ma[k, D_OUT:][None, :]).sum(axis=1, keepdims=True)
        sb = nbase + (zbk * ma[k, D_OUT:][None, :]).sum(axis=1, keepdims=True)
        sa = jnp.maximum(sa, 0.2 * sa)
        sb = jnp.maximum(sb, 0.2 * sb)
        m = jnp.maximum(sa, sb)
        pa = jnp.exp(sa - m)
        pb = jnp.exp(sb - m)
        outs.append((pa * zak + pb * zbk) / (pa + pb))
    mac = jnp.concatenate(outs, axis=1)

    res = jnp.dot(f, wres_ref[...], preferred_element_type=jnp.float32) \
        + bres_ref[...]
    al = alpha_ref[0, 0]
    out_ref[...] = al * mac + (1.0 - al) * res


def _post_call(f, fta, ftb, esa, esb, wrela, wrelb, wn, wres, bres, ma, alpha):
    row_spec = pl.BlockSpec((BN, D_IN), lambda i: (i, 0))
    ft_specs_a = [pl.BlockSpec((BN, D_OUT), lambda i, k=k: (k * NBLK + i, 0))
                  for k in range(K)]
    ft_specs_b = [pl.BlockSpec((BN, D_OUT), lambda i, k=k: (k * NBLK + i, 0))
                  for k in range(K)]
    es_specs_a = [pl.BlockSpec((BN, 16), lambda i, c=c: (c * NBLK + i, 0))
                  for c in range(NCORE)]
    es_specs_b = [pl.BlockSpec((BN, 16), lambda i, c=c: (c * NBLK + i, 0))
                  for c in range(NCORE)]
    w_spec = pl.BlockSpec((HID, HID), lambda i: (0, 0))
    b_spec = pl.BlockSpec((1, HID), lambda i: (0, 0))
    ma_spec = pl.BlockSpec((K, 2 * D_OUT), lambda i: (0, 0))
    al_spec = pl.BlockSpec((1, 1), lambda i: (0, 0))
    return pl.pallas_call(
        _post_body,
        grid=(NBLK,),
        in_specs=([row_spec] + [ft_specs_a[k] for k in range(K)]
                  + [ft_specs_b[k] for k in range(K)]
                  + es_specs_a + es_specs_b
                  + [w_spec, w_spec, w_spec, w_spec, b_spec, ma_spec, al_spec]),
        out_specs=row_spec,
        out_shape=jax.ShapeDtypeStruct((NP, D_IN), jnp.float32),
    )(f, fta, fta, fta, fta, ftb, ftb, ftb, ftb, esa, esa, esb, esb,
      wrela, wrelb, wn, wres, bres, ma, alpha)


def kernel(feat_user, feat_item, eidx_rates, eidx_rated_by, eidx_follows,
           eidx_similar, W_micro_user, W_micro_item, A_micro_user,
           A_micro_item, W_node_user, W_node_item, W_rel_rates,
           W_rel_rated_by, W_rel_follows, W_rel_similar, macro_attn,
           W_res_user, b_res_user, W_res_item, b_res_item, rw_user, rw_item):
    fu = jnp.pad(feat_user, ((0, NP - N), (0, 0)))
    fi = jnp.pad(feat_item, ((0, NP - N), (0, 0)))

    def pad_edges(eidx):
        src = jnp.pad(eidx[0].astype(jnp.int32), (0, EP - E),
                      constant_values=N)
        dst = jnp.pad(eidx[1].astype(jnp.int32), (0, EP - E),
                      constant_values=N)
        return src, dst

    src_rt, dst_rt = pad_edges(eidx_rates)
    src_rb, dst_rb = pad_edges(eidx_rated_by)
    src_fo, dst_fo = pad_edges(eidx_follows)
    src_si, dst_si = pad_edges(eidx_similar)

    hu, hi, elr = _pre_call(
        fu, fi, W_micro_user, W_micro_item, A_micro_user, A_micro_item)

    z8p = jnp.zeros((NP8, HID), jnp.float32)
    z4p = jnp.zeros((NP4, HID), jnp.float32)

    ee_rt, es_rt = _passA(0, 4)(src_rt, dst_rt, elr, z8p)
    ee_rb, es_rb = _passA(4, 8)(src_rb, dst_rb, elr, z8p)
    ee_fo, es_fo = _passA(0, 0)(src_fo, dst_fo, elr, z8p)
    ee_si, es_si = _passA(4, 12)(src_si, dst_si, elr, z8p)

    ft_rt = _passC()(src_rt, dst_rt, hu, ee_rt, z4p)
    ft_rb = _passC()(src_rb, dst_rb, hi, ee_rb, z4p)
    ft_fo = _passC()(src_fo, dst_fo, hu, ee_fo, z4p)
    ft_si = _passC()(src_si, dst_si, hi, ee_si, z4p)

    def unpack_es(es):
        return es.reshape(NCORE * NP, 16)

    def unpack_ft(ft):
        return ft.reshape(K * NP, D_OUT)

    es_rt, es_rb, es_fo, es_si = map(unpack_es, (es_rt, es_rb, es_fo, es_si))
    ft_rt, ft_rb, ft_fo, ft_si = map(unpack_ft, (ft_rt, ft_rb, ft_fo, ft_si))

    au = jax.nn.sigmoid(rw_user).reshape(1, 1)
    ai = jax.nn.sigmoid(rw_item).reshape(1, 1)

    out_user = _post_call(fu, ft_rb, ft_fo, es_rb, es_fo, W_rel_rated_by,
                          W_rel_follows, W_node_user, W_res_user,
                          b_res_user.reshape(1, HID), macro_attn, au)
    out_item = _post_call(fi, ft_rt, ft_si, es_rt, es_si, W_rel_rates,
                          W_rel_similar, W_node_item, W_res_item,
                          b_res_item.reshape(1, HID), macro_attn, ai)
    return out_user[:N], out_item[:N]


# speedup vs baseline: 10.6292x; 1.1717x over previous
import dataclasses
import functools

import jax
import jax.numpy as jnp
from jax import lax
from jax.experimental import pallas as pl
from jax.experimental.pallas import tpu as pltpu
from jax.experimental.pallas import tpu_sc as plsc

N = 50000
E = 150000
D_IN = 128
K = 4
D_OUT = 32
HID = K * D_OUT

NSC = 16
NCORE = 2
CHUNK = 64

NP = 50176
ROWS_PER_SUB = NP // NSC
ZROWS = 64

EP = 151552
EW_A = EP // (NCORE * NSC)
NCH_A = EW_A // CHUNK
EW_C = EP // NSC
CHUNK_C = 32
NCH_C = EW_C // CHUNK_C

BN = 512
NBLK = NP // BN

@functools.lru_cache(maxsize=None)
def _sc_params():
    cp = pltpu.CompilerParams()
    if "needs_layout_passes" in pltpu.CompilerParams.__dataclass_fields__:
        cp = dataclasses.replace(cp, needs_layout_passes=False)
    return cp


@functools.lru_cache(maxsize=None)
def _vector_mesh():
    return plsc.VectorSubcoreMesh(core_axis_name="c", subcore_axis_name="s",
                                  num_cores=NCORE, num_subcores=NSC)


def _pre_body(fu_ref, fi_ref, wmu_ref, wmi_ref, au_ref, ai_ref,
              hu_ref, hi_ref, elr_ref):
    fu = fu_ref[...]
    fi = fi_ref[...]
    hu = jnp.dot(fu, wmu_ref[...], preferred_element_type=jnp.float32)
    hi = jnp.dot(fi, wmi_ref[...], preferred_element_type=jnp.float32)
    hu_ref[...] = hu
    hi_ref[...] = hi
    au = au_ref[...]
    ai = ai_ref[...]

    def tab(h, a, lo):
        cols = []
        for k in range(K):
            hk = h[:, k * D_OUT:(k + 1) * D_OUT]
            ak = a[k, lo:lo + D_OUT][None, :]
            cols.append((hk * ak).sum(axis=1, keepdims=True))
        return jnp.concatenate(cols, axis=1)

    z8 = jnp.zeros((fu.shape[0], 8), jnp.float32)
    z96 = jnp.zeros((fu.shape[0], 96), jnp.float32)
    elr_ref[...] = jnp.concatenate(
        [tab(hu, au, 0), tab(hi, ai, 0), z8,
         tab(hu, au, D_OUT), tab(hi, au, D_OUT),
         tab(hu, ai, D_OUT), tab(hi, ai, D_OUT), z96], axis=1)


def _pre_call(fu, fi, wmu, wmi, au, ai):
    row_spec = pl.BlockSpec((BN, D_IN), lambda i: (i, 0))
    full_spec = pl.BlockSpec((D_IN, HID), lambda i: (0, 0))
    a_spec = pl.BlockSpec((K, 2 * D_OUT), lambda i: (0, 0))
    h_out = jax.ShapeDtypeStruct((NP, HID), jnp.float32)
    return pl.pallas_call(
        _pre_body,
        grid=(NBLK,),
        in_specs=[row_spec, row_spec, full_spec, full_spec, a_spec, a_spec],
        out_specs=[row_spec, row_spec, row_spec],
        out_shape=[h_out, h_out, h_out],
    )(fu, fi, wmu, wmi, au, ai)


NP8 = NP // 8
NP4 = NP // 4


def _passA_body(el_off, er_off, src_hbm, dst_hbm, elr_hbm, z_hbm, ee_hbm,
                esum_hbm, sbuf, dbuf, dbuf2, srcrow, dstrow, eebuf, obuf,
                esum_sh):
    c = lax.axis_index("c")
    s = lax.axis_index("s")
    w = s * NCORE + c

    @pl.when(s == 0)
    def _zero():
        pltpu.sync_copy(z_hbm, esum_sh)

    plsc.subcore_barrier()

    base = w * EW_A

    @pl.loop(0, NCH_A)
    def _edges(i):
        off = base + i * CHUNK
        pltpu.sync_copy(src_hbm.at[pl.ds(off, CHUNK)], sbuf)
        pltpu.sync_copy(dst_hbm.at[pl.ds(off, CHUNK)], dbuf)
        pltpu.sync_copy(elr_hbm.at[sbuf], srcrow)
        pltpu.sync_copy(elr_hbm.at[dbuf], dstrow)

        @pl.loop(0, CHUNK // 16)
        def _d2(g):
            dv = dbuf.at[pl.ds(g * 16, 16)][...]
            dbuf2.at[pl.ds(g * 16, 16)][...] = lax.shift_right_logical(dv, 3)

        @pl.loop(0, CHUNK)
        def _ee(j):
            lane = lax.iota(jnp.int32, 16) & 3
            jv = jnp.full((16,), j, jnp.int32)
            el = plsc.load_gather(srcrow, [jv, lane + el_off])
            er = plsc.load_gather(dstrow, [jv, lane + (16 + er_off)])
            x = el + er
            e = jnp.maximum(x, 0.2 * x)
            ee = jnp.exp(e)
            eebuf.at[j][...] = ee
            dv = plsc.load_gather(dbuf, [jv])
            grp = dv & 7
            for p in range(8):
                obuf.at[j, pl.ds(16 * p, 16)][...] = jnp.where(
                    grp == p, ee, 0.0)

        pltpu.sync_copy(eebuf, ee_hbm.at[pl.ds(off, CHUNK)])
        pltpu.sync_copy(obuf, esum_sh.at[dbuf2], add=True)

    plsc.subcore_barrier()

    @pl.when(s == 0)
    def _flush():
        pltpu.sync_copy(esum_sh, esum_hbm.at[pl.ds(c * NP8, NP8)])


@functools.lru_cache(maxsize=None)
def _passA(el_off, er_off):
    return pl.kernel(
        functools.partial(_passA_body, el_off, er_off),
        out_type=(jax.ShapeDtypeStruct((EP, 16), jnp.float32),
                  jax.ShapeDtypeStruct((NCORE * NP8, HID), jnp.float32)),
        mesh=_vector_mesh(),
        scratch_types=[
            pltpu.VMEM((CHUNK,), jnp.int32),
            pltpu.VMEM((CHUNK,), jnp.int32),
            pltpu.VMEM((CHUNK,), jnp.int32),
            pltpu.VMEM((CHUNK, HID), jnp.float32),
            pltpu.VMEM((CHUNK, HID), jnp.float32),
            pltpu.VMEM((CHUNK, 16), jnp.float32),
            pltpu.VMEM((CHUNK, HID), jnp.float32),
            pltpu.VMEM_SHARED((NP8, HID), jnp.float32),
        ],
        compiler_params=_sc_params(),
    )


def _passC_chunk(h, off, sbuf, dbuf, dbuf2, hbuf, eebuf, obuf, acc_sh):
    @pl.loop(0, CHUNK_C // 16)
    def _d2(g):
        dv = dbuf.at[pl.ds(g * 16, 16)][...]
        dbuf2.at[pl.ds(g * 16, 16)][...] = lax.shift_right_logical(dv, 2)

    @pl.loop(0, CHUNK_C)
    def _scale(j):
        jv = jnp.full((16,), j, jnp.int32)
        hv = jnp.full((16,), h, jnp.int32)
        a = plsc.load_gather(eebuf, [jv, hv])
        v0 = hbuf.at[j, pl.ds(32 * h, 16)][...] * a
        v1 = hbuf.at[j, pl.ds(32 * h + 16, 16)][...] * a
        dv = plsc.load_gather(dbuf, [jv])
        grp = dv & 3
        for p in range(4):
            obuf.at[j, pl.ds(32 * p, 16)][...] = jnp.where(grp == p, v0, 0.0)
            obuf.at[j, pl.ds(32 * p + 16, 16)][...] = jnp.where(
                grp == p, v1, 0.0)

    pltpu.sync_copy(obuf, acc_sh.at[dbuf2], add=True)


def _passC_body(src_hbm, dst_hbm, h_hbm, ee_hbm, z_hbm, ft_hbm,
                sbufA, dbufA, sbufB, dbufB, dbuf2, hbufA, hbufB,
                eebufA, eebufB, obuf, acc_sh,
                semAh, semAe, semBh, semBe):
    c = lax.axis_index("c")
    s = lax.axis_index("s")

    def fetch(off, sbuf, dbuf, hbuf, eebuf, semh, seme):
        pltpu.sync_copy(src_hbm.at[pl.ds(off, CHUNK_C)], sbuf)
        pltpu.sync_copy(dst_hbm.at[pl.ds(off, CHUNK_C)], dbuf)
        pltpu.async_copy(h_hbm.at[sbuf], hbuf, semh)
        pltpu.async_copy(ee_hbm.at[pl.ds(off, CHUNK_C)], eebuf, seme)

    for h in range(K):
        @pl.when(c == h // 2)
        def _head():
            @pl.when(s == 0)
            def _zero():
                pltpu.sync_copy(z_hbm, acc_sh)

            plsc.subcore_barrier()

            base = s * EW_C
            fetch(base, sbufA, dbufA, hbufA, eebufA, semAh, semAe)

            @pl.loop(0, NCH_C, step=2)
            def _edges(i):
                offA = base + i * CHUNK_C
                offB = offA + CHUNK_C
                fetch(offB, sbufB, dbufB, hbufB, eebufB, semBh, semBe)
                pltpu.make_async_copy(h_hbm.at[sbufA], hbufA, semAh).wait()
                pltpu.make_async_copy(
                    ee_hbm.at[pl.ds(offA, CHUNK_C)], eebufA, semAe).wait()
                _passC_chunk(h, offA, sbufA, dbufA, dbuf2, hbufA, eebufA,
                             obuf, acc_sh)

                @pl.when(i + 2 < NCH_C)
                def _prefA():
                    fetch(offB + CHUNK_C, sbufA, dbufA, hbufA, eebufA,
                          semAh, semAe)

                pltpu.make_async_copy(h_hbm.at[sbufB], hbufB, semBh).wait()
                pltpu.make_async_copy(
                    ee_hbm.at[pl.ds(offB, CHUNK_C)], eebufB, semBe).wait()
                _passC_chunk(h, offB, sbufB, dbufB, dbuf2, hbufB, eebufB,
                             obuf, acc_sh)

            plsc.subcore_barrier()

            @pl.when(s == 0)
            def _flush():
                pltpu.sync_copy(acc_sh, ft_hbm.at[pl.ds(h * NP4, NP4)])

            plsc.subcore_barrier()


@functools.lru_cache(maxsize=None)
def _passC():
    return pl.kernel(
        _passC_body,
        out_type=jax.ShapeDtypeStruct((K * NP4, HID), jnp.float32),
        mesh=_vector_mesh(),
        scratch_types=[
            pltpu.VMEM((CHUNK_C,), jnp.int32),
            pltpu.VMEM((CHUNK_C,), jnp.int32),
            pltpu.VMEM((CHUNK_C,), jnp.int32),
            pltpu.VMEM((CHUNK_C,), jnp.int32),
            pltpu.VMEM((CHUNK_C,), jnp.int32),
            pltpu.VMEM((CHUNK_C, HID), jnp.float32),
            pltpu.VMEM((CHUNK_C, HID), jnp.float32),
            pltpu.VMEM((CHUNK_C, 16), jnp.float32),
            pltpu.VMEM((CHUNK_C, 16), jnp.float32),
            pltpu.VMEM((CHUNK_C, HID), jnp.float32),
            pltpu.VMEM_SHARED((NP4, HID), jnp.float32),
            pltpu.SemaphoreType.DMA,
            pltpu.SemaphoreType.DMA,
            pltpu.SemaphoreType.DMA,
            pltpu.SemaphoreType.DMA,
        ],
        compiler_params=_sc_params(),
    )


def _post_body(f_ref, fta0, fta1, fta2, fta3, ftb0, ftb1, ftb2, ftb3,
               esa0, esa1, esb0, esb1, wrela_ref, wrelb_ref, wn_ref,
               wres_ref, bres_ref, ma_ref, alpha_ref, out_ref):
    f = f_ref[...]

    def zrel(ft_refs, es0, es1, wrel_ref):
        esum = es0[...] + es1[...]
        cols = []
        for k in range(K):
            d = esum[:, k:k + 1]
            d = jnp.where(d > 0.0, d, 1.0)
            cols.append(jnp.maximum(ft_refs[k][...] / d, 0.0))
        r = jnp.concatenate(cols, axis=1)
        return jnp.dot(r, wrel_ref[...], preferred_element_type=jnp.float32)

    za = zrel((fta0, fta1, fta2, fta3), esa0, esa1, wrela_ref)
    zb = zrel((ftb0, ftb1, ftb2, ftb3), esb0, esb1, wrelb_ref)
    node = jnp.dot(f, wn_ref[...], preferred_element_type=jnp.float32)
    ma = ma_ref[...]

    outs = []
    for k in range(K):
        sl = slice(k * D_OUT, (k + 1) * D_OUT)
        nk, zak, zbk = node[:, sl], za[:, sl], zb[:, sl]
        nbase = (nk * ma[k, :D_OUT][None, :]).sum(axis=1, keepdims=True)
        sa = nbase + (zak * ma[k, D_OUT:][None, :]).sum(axis=1, keepdims=True)
        sb = nbase + (zbk * ma[k, D_OUT:][None, :]).sum(axis=1, keepdims=True)
        sa = jnp.maximum(sa, 0.2 * sa)
        sb = jnp.maximum(sb, 0.2 * sb)
        m = jnp.maximum(sa, sb)
        pa = jnp.exp(sa - m)
        pb = jnp.exp(sb - m)
        outs.append((pa * zak + pb * zbk) / (pa + pb))
    mac = jnp.concatenate(outs, axis=1)

    res = jnp.dot(f, wres_ref[...], preferred_element_type=jnp.float32) \
        + bres_ref[...]
    al = alpha_ref[0, 0]
    out_ref[...] = al * mac + (1.0 - al) * res


def _post_call(f, fta, ftb, esa, esb, wrela, wrelb, wn, wres, bres, ma, alpha):
    row_spec = pl.BlockSpec((BN, D_IN), lambda i: (i, 0))
    ft_specs_a = [pl.BlockSpec((BN, D_OUT), lambda i, k=k: (k * NBLK + i, 0))
                  for k in range(K)]
    ft_specs_b = [pl.BlockSpec((BN, D_OUT), lambda i, k=k: (k * NBLK + i, 0))
                  for k in range(K)]
    es_specs_a = [pl.BlockSpec((BN, 16), lambda i, c=c: (c * NBLK + i, 0))
                  for c in range(NCORE)]
    es_specs_b = [pl.BlockSpec((BN, 16), lambda i, c=c: (c * NBLK + i, 0))
                  for c in range(NCORE)]
    w_spec = pl.BlockSpec((HID, HID), lambda i: (0, 0))
    b_spec = pl.BlockSpec((1, HID), lambda i: (0, 0))
    ma_spec = pl.BlockSpec((K, 2 * D_OUT), lambda i: (0, 0))
    al_spec = pl.BlockSpec((1, 1), lambda i: (0, 0))
    return pl.pallas_call(
        _post_body,
        grid=(NBLK,),
        in_specs=([row_spec] + [ft_specs_a[k] for k in range(K)]
                  + [ft_specs_b[k] for k in range(K)]
                  + es_specs_a + es_specs_b
                  + [w_spec, w_spec, w_spec, w_spec, b_spec, ma_spec, al_spec]),
        out_specs=row_spec,
        out_shape=jax.ShapeDtypeStruct((NP, D_IN), jnp.float32),
    )(f, fta, fta, fta, fta, ftb, ftb, ftb, ftb, esa, esa, esb, esb,
      wrela, wrelb, wn, wres, bres, ma, alpha)


def kernel(feat_user, feat_item, eidx_rates, eidx_rated_by, eidx_follows,
           eidx_similar, W_micro_user, W_micro_item, A_micro_user,
           A_micro_item, W_node_user, W_node_item, W_rel_rates,
           W_rel_rated_by, W_rel_follows, W_rel_similar, macro_attn,
           W_res_user, b_res_user, W_res_item, b_res_item, rw_user, rw_item):
    fu = jnp.pad(feat_user, ((0, NP - N), (0, 0)))
    fi = jnp.pad(feat_item, ((0, NP - N), (0, 0)))

    def pad_edges(eidx):
        src = jnp.pad(eidx[0].astype(jnp.int32), (0, EP - E),
                      constant_values=N)
        dst = jnp.pad(eidx[1].astype(jnp.int32), (0, EP - E),
                      constant_values=N)
        return src, dst

    src_rt, dst_rt = pad_edges(eidx_rates)
    src_rb, dst_rb = pad_edges(eidx_rated_by)
    src_fo, dst_fo = pad_edges(eidx_follows)
    src_si, dst_si = pad_edges(eidx_similar)

    hu, hi, elr = _pre_call(
        fu, fi, W_micro_user, W_micro_item, A_micro_user, A_micro_item)

    z8p = jnp.zeros((NP8, HID), jnp.float32)
    z4p = jnp.zeros((NP4, HID), jnp.float32)

    ee_rt, es_rt = _passA(0, 4)(src_rt, dst_rt, elr, z8p)
    ee_rb, es_rb = _passA(4, 8)(src_rb, dst_rb, elr, z8p)
    ee_fo, es_fo = _passA(0, 0)(src_fo, dst_fo, elr, z8p)
    ee_si, es_si = _passA(4, 12)(src_si, dst_si, elr, z8p)

    ft_rt = _passC()(src_rt, dst_rt, hu, ee_rt, z4p)
    ft_rb = _passC()(src_rb, dst_rb, hi, ee_rb, z4p)
    ft_fo = _passC()(src_fo, dst_fo, hu, ee_fo, z4p)
    ft_si = _passC()(src_si, dst_si, hi, ee_si, z4p)

    def unpack_es(es):
        return es.reshape(NCORE * NP, 16)

    def unpack_ft(ft):
        return ft.reshape(K * NP, D_OUT)

    es_rt, es_rb, es_fo, es_si = map(unpack_es, (es_rt, es_rb, es_fo, es_si))
    ft_rt, ft_rb, ft_fo, ft_si = map(unpack_ft, (ft_rt, ft_rb, ft_fo, ft_si))

    au = jax.nn.sigmoid(rw_user).reshape(1, 1)
    ai = jax.nn.sigmoid(rw_item).reshape(1, 1)

    out_user = _post_call(fu, ft_rb, ft_fo, es_rb, es_fo, W_rel_rated_by,
                          W_rel_follows, W_node_user, W_res_user,
                          b_res_user.reshape(1, HID), macro_attn, au)
    out_item = _post_call(fi, ft_rt, ft_si, es_rt, es_si, W_rel_rates,
                          W_rel_similar, W_node_item, W_res_item,
                          b_res_item.reshape(1, HID), macro_attn, ai)
    return out_user[:N], out_item[:N]


# speedup vs baseline: 11.7300x; 1.1036x over previous
import dataclasses
import functools

import jax
import jax.numpy as jnp
from jax import lax
from jax.experimental import pallas as pl
from jax.experimental.pallas import tpu as pltpu
from jax.experimental.pallas import tpu_sc as plsc

N = 50000
E = 150000
D_IN = 128
K = 4
D_OUT = 32
HID = K * D_OUT

NSC = 16
NCORE = 2
CHUNK = 64

NP = 50176
ROWS_PER_SUB = NP // NSC
ZROWS = 64

EP = 151552
EW_A = EP // (NCORE * NSC)
NCH_A = EW_A // CHUNK
EW_C = EP // NSC
CHUNK_C = 32
NCH_C = EW_C // CHUNK_C

BN = 512
NBLK = NP // BN

@functools.lru_cache(maxsize=None)
def _sc_params():
    cp = pltpu.CompilerParams()
    if "needs_layout_passes" in pltpu.CompilerParams.__dataclass_fields__:
        cp = dataclasses.replace(cp, needs_layout_passes=False)
    return cp


@functools.lru_cache(maxsize=None)
def _vector_mesh():
    return plsc.VectorSubcoreMesh(core_axis_name="c", subcore_axis_name="s",
                                  num_cores=NCORE, num_subcores=NSC)


def _pre_body(fu_ref, fi_ref, wmu_ref, wmi_ref, au_ref, ai_ref,
              hu_ref, hi_ref, elr_ref):
    fu = fu_ref[...]
    fi = fi_ref[...]
    hu = jnp.dot(fu, wmu_ref[...], preferred_element_type=jnp.float32)
    hi = jnp.dot(fi, wmi_ref[...], preferred_element_type=jnp.float32)
    hu_ref[...] = hu
    hi_ref[...] = hi
    au = au_ref[...]
    ai = ai_ref[...]

    def tab(h, a, lo):
        cols = []
        for k in range(K):
            hk = h[:, k * D_OUT:(k + 1) * D_OUT]
            ak = a[k, lo:lo + D_OUT][None, :]
            cols.append((hk * ak).sum(axis=1, keepdims=True))
        return jnp.concatenate(cols, axis=1)

    z8 = jnp.zeros((fu.shape[0], 8), jnp.float32)
    z96 = jnp.zeros((fu.shape[0], 96), jnp.float32)
    elr_ref[...] = jnp.concatenate(
        [tab(hu, au, 0), tab(hi, ai, 0), z8,
         tab(hu, au, D_OUT), tab(hi, au, D_OUT),
         tab(hu, ai, D_OUT), tab(hi, ai, D_OUT), z96], axis=1)


def _pre_call(fu, fi, wmu, wmi, au, ai):
    row_spec = pl.BlockSpec((BN, D_IN), lambda i: (i, 0))
    full_spec = pl.BlockSpec((D_IN, HID), lambda i: (0, 0))
    a_spec = pl.BlockSpec((K, 2 * D_OUT), lambda i: (0, 0))
    h_out = jax.ShapeDtypeStruct((NP, HID), jnp.float32)
    return pl.pallas_call(
        _pre_body,
        grid=(NBLK,),
        in_specs=[row_spec, row_spec, full_spec, full_spec, a_spec, a_spec],
        out_specs=[row_spec, row_spec, row_spec],
        out_shape=[h_out, h_out, h_out],
    )(fu, fi, wmu, wmi, au, ai)


NP8 = NP // 8
NP4 = NP // 4


def _passA_chunk(el_off, er_off, off, dbuf, dbuf2, srcrow, dstrow, eebuf,
                 obuf, ee_hbm, esum_sh):
    @pl.loop(0, CHUNK // 16)
    def _d2(g):
        dv = dbuf.at[pl.ds(g * 16, 16)][...]
        dbuf2.at[pl.ds(g * 16, 16)][...] = lax.shift_right_logical(dv, 3)

    @pl.loop(0, CHUNK)
    def _ee(j):
        lane = lax.iota(jnp.int32, 16) & 3
        jv = jnp.full((16,), j, jnp.int32)
        el = plsc.load_gather(srcrow, [jv, lane + el_off])
        er = plsc.load_gather(dstrow, [jv, lane + (16 + er_off)])
        x = el + er
        e = jnp.maximum(x, 0.2 * x)
        ee = jnp.exp(e)
        eebuf.at[j][...] = ee
        dv = plsc.load_gather(dbuf, [jv])
        grp = dv & 7
        for p in range(8):
            obuf.at[j, pl.ds(16 * p, 16)][...] = jnp.where(grp == p, ee, 0.0)

    pltpu.sync_copy(eebuf, ee_hbm.at[pl.ds(off, CHUNK)])
    pltpu.sync_copy(obuf, esum_sh.at[dbuf2], add=True)


def _passA_body(el_off, er_off, src_hbm, dst_hbm, elr_hbm, z_hbm, ee_hbm,
                esum_hbm, sbufA, dbufA, sbufB, dbufB, dbuf2,
                srcrowA, dstrowA, srcrowB, dstrowB, eebuf, obuf, esum_sh,
                semAs, semAd, semBs, semBd):
    c = lax.axis_index("c")
    s = lax.axis_index("s")
    w = s * NCORE + c

    @pl.when(s == 0)
    def _zero():
        pltpu.sync_copy(z_hbm, esum_sh)

    plsc.subcore_barrier()

    base = w * EW_A

    def fetch(off, sbuf, dbuf, srcrow, dstrow, sems, semd):
        pltpu.sync_copy(src_hbm.at[pl.ds(off, CHUNK)], sbuf)
        pltpu.sync_copy(dst_hbm.at[pl.ds(off, CHUNK)], dbuf)
        pltpu.async_copy(elr_hbm.at[sbuf], srcrow, sems)
        pltpu.async_copy(elr_hbm.at[dbuf], dstrow, semd)

    fetch(base, sbufA, dbufA, srcrowA, dstrowA, semAs, semAd)

    @pl.loop(0, NCH_A, step=2)
    def _edges(i):
        offA = base + i * CHUNK
        offB = offA + CHUNK
        fetch(offB, sbufB, dbufB, srcrowB, dstrowB, semBs, semBd)
        pltpu.make_async_copy(elr_hbm.at[sbufA], srcrowA, semAs).wait()
        pltpu.make_async_copy(elr_hbm.at[dbufA], dstrowA, semAd).wait()
        _passA_chunk(el_off, er_off, offA, dbufA, dbuf2, srcrowA, dstrowA,
                     eebuf, obuf, ee_hbm, esum_sh)

        @pl.when(i + 2 < NCH_A)
        def _prefA():
            fetch(offB + CHUNK, sbufA, dbufA, srcrowA, dstrowA, semAs, semAd)

        pltpu.make_async_copy(elr_hbm.at[sbufB], srcrowB, semBs).wait()
        pltpu.make_async_copy(elr_hbm.at[dbufB], dstrowB, semBd).wait()
        _passA_chunk(el_off, er_off, offB, dbufB, dbuf2, srcrowB, dstrowB,
                     eebuf, obuf, ee_hbm, esum_sh)

    plsc.subcore_barrier()

    @pl.when(s == 0)
    def _flush():
        pltpu.sync_copy(esum_sh, esum_hbm.at[pl.ds(c * NP8, NP8)])


@functools.lru_cache(maxsize=None)
def _passA(el_off, er_off):
    return pl.kernel(
        functools.partial(_passA_body, el_off, er_off),
        out_type=(jax.ShapeDtypeStruct((EP, 16), jnp.float32),
                  jax.ShapeDtypeStruct((NCORE * NP8, HID), jnp.float32)),
        mesh=_vector_mesh(),
        scratch_types=[
            pltpu.VMEM((CHUNK,), jnp.int32),
            pltpu.VMEM((CHUNK,), jnp.int32),
            pltpu.VMEM((CHUNK,), jnp.int32),
            pltpu.VMEM((CHUNK,), jnp.int32),
            pltpu.VMEM((CHUNK,), jnp.int32),
            pltpu.VMEM((CHUNK, HID), jnp.float32),
            pltpu.VMEM((CHUNK, HID), jnp.float32),
            pltpu.VMEM((CHUNK, HID), jnp.float32),
            pltpu.VMEM((CHUNK, HID), jnp.float32),
            pltpu.VMEM((CHUNK, 16), jnp.float32),
            pltpu.VMEM((CHUNK, HID), jnp.float32),
            pltpu.VMEM_SHARED((NP8, HID), jnp.float32),
            pltpu.SemaphoreType.DMA,
            pltpu.SemaphoreType.DMA,
            pltpu.SemaphoreType.DMA,
            pltpu.SemaphoreType.DMA,
        ],
        compiler_params=_sc_params(),
    )


def _passC_chunk(h, off, sbuf, dbuf, dbuf2, hbuf, eebuf, obuf, acc_sh):
    @pl.loop(0, CHUNK_C // 16)
    def _d2(g):
        dv = dbuf.at[pl.ds(g * 16, 16)][...]
        dbuf2.at[pl.ds(g * 16, 16)][...] = lax.shift_right_logical(dv, 2)

    @pl.loop(0, CHUNK_C)
    def _scale(j):
        jv = jnp.full((16,), j, jnp.int32)
        hv = jnp.full((16,), h, jnp.int32)
        a = plsc.load_gather(eebuf, [jv, hv])
        v0 = hbuf.at[j, pl.ds(32 * h, 16)][...] * a
        v1 = hbuf.at[j, pl.ds(32 * h + 16, 16)][...] * a
        dv = plsc.load_gather(dbuf, [jv])
        grp = dv & 3
        for p in range(4):
            obuf.at[j, pl.ds(32 * p, 16)][...] = jnp.where(grp == p, v0, 0.0)
            obuf.at[j, pl.ds(32 * p + 16, 16)][...] = jnp.where(
                grp == p, v1, 0.0)

    pltpu.sync_copy(obuf, acc_sh.at[dbuf2], add=True)


def _passC_body(src_hbm, dst_hbm, h_hbm, ee_hbm, z_hbm, ft_hbm,
                sbufA, dbufA, sbufB, dbufB, dbuf2, hbufA, hbufB,
                eebufA, eebufB, obuf, acc_sh,
                semAh, semAe, semBh, semBe):
    c = lax.axis_index("c")
    s = lax.axis_index("s")

    def fetch(off, sbuf, dbuf, hbuf, eebuf, semh, seme):
        pltpu.sync_copy(src_hbm.at[pl.ds(off, CHUNK_C)], sbuf)
        pltpu.sync_copy(dst_hbm.at[pl.ds(off, CHUNK_C)], dbuf)
        pltpu.async_copy(h_hbm.at[sbuf], hbuf, semh)
        pltpu.async_copy(ee_hbm.at[pl.ds(off, CHUNK_C)], eebuf, seme)

    for h in range(K):
        @pl.when(c == h // 2)
        def _head():
            @pl.when(s == 0)
            def _zero():
                pltpu.sync_copy(z_hbm, acc_sh)

            plsc.subcore_barrier()

            base = s * EW_C
            fetch(base, sbufA, dbufA, hbufA, eebufA, semAh, semAe)

            @pl.loop(0, NCH_C, step=2)
            def _edges(i):
                offA = base + i * CHUNK_C
                offB = offA + CHUNK_C
                fetch(offB, sbufB, dbufB, hbufB, eebufB, semBh, semBe)
                pltpu.make_async_copy(h_hbm.at[sbufA], hbufA, semAh).wait()
                pltpu.make_async_copy(
                    ee_hbm.at[pl.ds(offA, CHUNK_C)], eebufA, semAe).wait()
                _passC_chunk(h, offA, sbufA, dbufA, dbuf2, hbufA, eebufA,
                             obuf, acc_sh)

                @pl.when(i + 2 < NCH_C)
                def _prefA():
                    fetch(offB + CHUNK_C, sbufA, dbufA, hbufA, eebufA,
                          semAh, semAe)

                pltpu.make_async_copy(h_hbm.at[sbufB], hbufB, semBh).wait()
                pltpu.make_async_copy(
                    ee_hbm.at[pl.ds(offB, CHUNK_C)], eebufB, semBe).wait()
                _passC_chunk(h, offB, sbufB, dbufB, dbuf2, hbufB, eebufB,
                             obuf, acc_sh)

            plsc.subcore_barrier()

            @pl.when(s == 0)
            def _flush():
                pltpu.sync_copy(acc_sh, ft_hbm.at[pl.ds(h * NP4, NP4)])

            plsc.subcore_barrier()


@functools.lru_cache(maxsize=None)
def _passC():
    return pl.kernel(
        _passC_body,
        out_type=jax.ShapeDtypeStruct((K * NP4, HID), jnp.float32),
        mesh=_vector_mesh(),
        scratch_types=[
            pltpu.VMEM((CHUNK_C,), jnp.int32),
            pltpu.VMEM((CHUNK_C,), jnp.int32),
            pltpu.VMEM((CHUNK_C,), jnp.int32),
            pltpu.VMEM((CHUNK_C,), jnp.int32),
            pltpu.VMEM((CHUNK_C,), jnp.int32),
            pltpu.VMEM((CHUNK_C, HID), jnp.float32),
            pltpu.VMEM((CHUNK_C, HID), jnp.float32),
            pltpu.VMEM((CHUNK_C, 16), jnp.float32),
            pltpu.VMEM((CHUNK_C, 16), jnp.float32),
            pltpu.VMEM((CHUNK_C, HID), jnp.float32),
            pltpu.VMEM_SHARED((NP4, HID), jnp.float32),
            pltpu.SemaphoreType.DMA,
            pltpu.SemaphoreType.DMA,
            pltpu.SemaphoreType.DMA,
            pltpu.SemaphoreType.DMA,
        ],
        compiler_params=_sc_params(),
    )


def _post_body(f_ref, fta0, fta1, fta2, fta3, ftb0, ftb1, ftb2, ftb3,
               esa0, esa1, esb0, esb1, wrela_ref, wrelb_ref, wn_ref,
               wres_ref, bres_ref, ma_ref, alpha_ref, out_ref):
    f = f_ref[...]

    def zrel(ft_refs, es0, es1, wrel_ref):
        esum = es0[...] + es1[...]
        cols = []
        for k in range(K):
            d = esum[:, k:k + 1]
            d = jnp.where(d > 0.0, d, 1.0)
            cols.append(jnp.maximum(ft_refs[k][...] / d, 0.0))
        r = jnp.concatenate(cols, axis=1)
        return jnp.dot(r, wrel_ref[...], preferred_element_type=jnp.float32)

    za = zrel((fta0, fta1, fta2, fta3), esa0, esa1, wrela_ref)
    zb = zrel((ftb0, ftb1, ftb2, ftb3), esb0, esb1, wrelb_ref)
    node = jnp.dot(f, wn_ref[...], preferred_element_type=jnp.float32)
    ma = ma_ref[...]

    outs = []
    for k in range(K):
        sl = slice(k * D_OUT, (k + 1) * D_OUT)
        nk, zak, zbk = node[:, sl], za[:, sl], zb[:, sl]
        nbase = (nk * ma[k, :D_OUT][None, :]).sum(axis=1, keepdims=True)
        sa = nbase + (zak * ma[k, D_OUT:][None, :]).sum(axis=1, keepdims=True)
        sb = nbase + (zbk * ma[k, D_OUT:][None, :]).sum(axis=1, keepdims=True)
        sa = jnp.maximum(sa, 0.2 * sa)
        sb = jnp.maximum(sb, 0.2 * sb)
        m = jnp.maximum(sa, sb)
        pa = jnp.exp(sa - m)
        pb = jnp.exp(sb - m)
        outs.append((pa * zak + pb * zbk) / (pa + pb))
    mac = jnp.concatenate(outs, axis=1)

    res = jnp.dot(f, wres_ref[...], preferred_element_type=jnp.float32) \
        + bres_ref[...]
    al = alpha_ref[0, 0]
    out_ref[...] = al * mac + (1.0 - al) * res


def _post_call(f, fta, ftb, esa, esb, wrela, wrelb, wn, wres, bres, ma, alpha):
    row_spec = pl.BlockSpec((BN, D_IN), lambda i: (i, 0))
    ft_specs_a = [pl.BlockSpec((BN, D_OUT), lambda i, k=k: (k * NBLK + i, 0))
                  for k in range(K)]
    ft_specs_b = [pl.BlockSpec((BN, D_OUT), lambda i, k=k: (k * NBLK + i, 0))
                  for k in range(K)]
    es_specs_a = [pl.BlockSpec((BN, 16), lambda i, c=c: (c * NBLK + i, 0))
                  for c in range(NCORE)]
    es_specs_b = [pl.BlockSpec((BN, 16), lambda i, c=c: (c * NBLK + i, 0))
                  for c in range(NCORE)]
    w_spec = pl.BlockSpec((HID, HID), lambda i: (0, 0))
    b_spec = pl.BlockSpec((1, HID), lambda i: (0, 0))
    ma_spec = pl.BlockSpec((K, 2 * D_OUT), lambda i: (0, 0))
    al_spec = pl.BlockSpec((1, 1), lambda i: (0, 0))
    return pl.pallas_call(
        _post_body,
        grid=(NBLK,),
        in_specs=([row_spec] + [ft_specs_a[k] for k in range(K)]
                  + [ft_specs_b[k] for k in range(K)]
                  + es_specs_a + es_specs_b
                  + [w_spec, w_spec, w_spec, w_spec, b_spec, ma_spec, al_spec]),
        out_specs=row_spec,
        out_shape=jax.ShapeDtypeStruct((NP, D_IN), jnp.float32),
    )(f, fta, fta, fta, fta, ftb, ftb, ftb, ftb, esa, esa, esb, esb,
      wrela, wrelb, wn, wres, bres, ma, alpha)


def kernel(feat_user, feat_item, eidx_rates, eidx_rated_by, eidx_follows,
           eidx_similar, W_micro_user, W_micro_item, A_micro_user,
           A_micro_item, W_node_user, W_node_item, W_rel_rates,
           W_rel_rated_by, W_rel_follows, W_rel_similar, macro_attn,
           W_res_user, b_res_user, W_res_item, b_res_item, rw_user, rw_item):
    fu = jnp.pad(feat_user, ((0, NP - N), (0, 0)))
    fi = jnp.pad(feat_item, ((0, NP - N), (0, 0)))

    def pad_edges(eidx):
        src = jnp.pad(eidx[0].astype(jnp.int32), (0, EP - E),
                      constant_values=N)
        dst = jnp.pad(eidx[1].astype(jnp.int32), (0, EP - E),
                      constant_values=N)
        return src, dst

    src_rt, dst_rt = pad_edges(eidx_rates)
    src_rb, dst_rb = pad_edges(eidx_rated_by)
    src_fo, dst_fo = pad_edges(eidx_follows)
    src_si, dst_si = pad_edges(eidx_similar)

    hu, hi, elr = _pre_call(
        fu, fi, W_micro_user, W_micro_item, A_micro_user, A_micro_item)

    z8p = jnp.zeros((NP8, HID), jnp.float32)
    z4p = jnp.zeros((NP4, HID), jnp.float32)

    ee_rt, es_rt = _passA(0, 4)(src_rt, dst_rt, elr, z8p)
    ee_rb, es_rb = _passA(4, 8)(src_rb, dst_rb, elr, z8p)
    ee_fo, es_fo = _passA(0, 0)(src_fo, dst_fo, elr, z8p)
    ee_si, es_si = _passA(4, 12)(src_si, dst_si, elr, z8p)

    ft_rt = _passC()(src_rt, dst_rt, hu, ee_rt, z4p)
    ft_rb = _passC()(src_rb, dst_rb, hi, ee_rb, z4p)
    ft_fo = _passC()(src_fo, dst_fo, hu, ee_fo, z4p)
    ft_si = _passC()(src_si, dst_si, hi, ee_si, z4p)

    def unpack_es(es):
        return es.reshape(NCORE * NP, 16)

    def unpack_ft(ft):
        return ft.reshape(K * NP, D_OUT)

    es_rt, es_rb, es_fo, es_si = map(unpack_es, (es_rt, es_rb, es_fo, es_si))
    ft_rt, ft_rb, ft_fo, ft_si = map(unpack_ft, (ft_rt, ft_rb, ft_fo, ft_si))

    au = jax.nn.sigmoid(rw_user).reshape(1, 1)
    ai = jax.nn.sigmoid(rw_item).reshape(1, 1)

    out_user = _post_call(fu, ft_rb, ft_fo, es_rb, es_fo, W_rel_rated_by,
                          W_rel_follows, W_node_user, W_res_user,
                          b_res_user.reshape(1, HID), macro_attn, au)
    out_item = _post_call(fi, ft_rt, ft_si, es_rt, es_si, W_rel_rates,
                          W_rel_similar, W_node_item, W_res_item,
                          b_res_item.reshape(1, HID), macro_attn, ai)
    return out_user[:N], out_item[:N]
